# baseline (speedup 1.0000x reference)
import functools

import jax
import jax.numpy as jnp
from jax import lax
from jax.experimental import pallas as pl
from jax.experimental.pallas import tpu as pltpu
from jax.experimental.pallas import tpu_sc as plsc

BATCH = 16384
MAX_LEN = 50
EMBED_DIM = 32
TOTAL = BATCH * MAX_LEN

_info = plsc.get_sparse_core_info()
_NC = _info.num_cores
_NS = _info.num_subcores
_NW = _NC * _NS
_B_PER_W = TOTAL // _NW
GROUP = 512
N_GROUPS = _B_PER_W // GROUP
NBUF = 4
_MAIN = (N_GROUPS // NBUF) * NBUF
_BL = 129

_mesh = plsc.VectorSubcoreMesh(core_axis_name="c", subcore_axis_name="s")


@functools.partial(
    pl.kernel,
    mesh=plsc.VectorSubcoreMesh(core_axis_name="c", subcore_axis_name="s"),
    out_type=jax.ShapeDtypeStruct((TOTAL,), jnp.int32),
    scratch_types=[
        pltpu.VMEM((8, 512), jnp.int32),
        pltpu.VMEM((8, 512), jnp.int32),
        pltpu.SemaphoreType.DMA,
        pltpu.SemaphoreType.DMA,
    ],
)
def _detile_idx(ids_hbm, out_hbm, v0, v1, sem0, sem1):
    wid = lax.axis_index("s") * _NC + lax.axis_index("c")
    b0 = wid * 512
    bufs = (v0, v1)
    sems = (sem0, sem1)

    def rows_of(u):
        return 8 if (u + 1) * 8 <= MAX_LEN else MAX_LEN - u * 8

    def start_in(u):
        r = rows_of(u)
        return pltpu.async_copy(
            ids_hbm.at[pl.ds(u * 8, r), pl.ds(b0, 512)],
            bufs[u % 2].at[pl.ds(0, r)], sems[u % 2])

    start_in(0)
    for u in range(7):
        if u + 1 < 7:
            start_in(u + 1)
        r = rows_of(u)
        pltpu.make_async_copy(
            ids_hbm.at[pl.ds(u * 8, r), pl.ds(b0, 512)],
            bufs[u % 2].at[pl.ds(0, r)], sems[u % 2]).wait()
        for tl in range(8):
            t = u * 8 + tl
            if t < MAX_LEN:
                pltpu.sync_copy(
                    bufs[u % 2].at[tl], out_hbm.at[pl.ds(t * BATCH + b0, 512)])


_scratch = (
    [pltpu.VMEM((GROUP,), jnp.int32) for _ in range(NBUF)]
    + [pltpu.VMEM((GROUP, EMBED_DIM), jnp.float32) for _ in range(NBUF)]
    + [pltpu.VMEM((16, 8, _BL), jnp.float32) for _ in range(2)]
    + [pltpu.SemaphoreType.DMA for _ in range(2 * NBUF + 2)]
)


@functools.partial(
    pl.kernel,
    mesh=_mesh,
    out_type=jax.ShapeDtypeStruct((MAX_LEN, 4, 128, 8, 128), jnp.float32),
    scratch_types=_scratch,
    compiler_params=pltpu.CompilerParams(
        use_tc_tiling_on_sc=False, needs_layout_passes=False),
)
def _gather_kernel(table_hbm, idx_hbm, out_hbm, *scratch):
    idx_v = scratch[0:NBUF]
    rows_v = scratch[NBUF:2 * NBUF]
    tr_v = scratch[2 * NBUF:2 * NBUF + 2]
    isem = scratch[2 * NBUF + 2:3 * NBUF + 2]
    gsem = scratch[3 * NBUF + 2:4 * NBUF + 2]
    ssem = scratch[4 * NBUF + 2:4 * NBUF + 4]

    wid = lax.axis_index("s") * _NC + lax.axis_index("c")
    base = wid * _B_PER_W
    lane = lax.iota(jnp.int32, 16)
    chbh_c = [(h * 2 + lane // 8) * 4 for h in range(2)]
    cl_c = lane % 8

    def tb_of(g, b):
        p0 = base + g * GROUP
        t = p0 >> 14
        bh0 = (p0 & 16383) >> 7
        return t, bh0

    def idx_src(g):
        return idx_hbm.at[pl.ds(base + g * GROUP, GROUP)]

    def start_idx(g, b):
        return pltpu.async_copy(idx_src(g), idx_v[b], isem[b])

    def wait_idx(g, b):
        pltpu.make_async_copy(idx_src(g), idx_v[b], isem[b]).wait()

    def start_gather(b):
        return pltpu.async_copy(table_hbm.at[idx_v[b]], rows_v[b], gsem[b])

    def wait_gather(b):
        pltpu.make_async_copy(table_hbm.at[idx_v[b]], rows_v[b], gsem[b]).wait()

    def store_pairs(g, tb):
        t, bh0 = tb_of(g, None)
        return [
            (tr_v[tb].at[pl.ds(ch * 4, 4), :, pl.ds(0, 128)],
             out_hbm.at[t, ch, pl.ds(bh0, 4), :, :])
            for ch in range(4)
        ]

    def start_store(g, tb):
        for src, dst in store_pairs(g, tb):
            pltpu.async_copy(src, dst, ssem[tb])

    def wait_store(g, tb):
        for src, dst in store_pairs(g, tb):
            pltpu.make_async_copy(src, dst, ssem[tb]).wait()

    def transpose(b, tb):
        def body(u0, carry):
            for du in range(32):
                u = u0 * 32 + du
                tk = u >> 1
                h = du & 1
                bh = tk >> 7
                bl = tk & 127
                v = rows_v[b][tk, pl.ds(h * 16, 16)]
                plsc.store_scatter(
                    tr_v[tb],
                    [chbh_c[h] + bh, cl_c, jnp.full((16,), 0, jnp.int32) + bl],
                    v)
            return carry
        lax.fori_loop(0, GROUP * 2 // 32, body, 0, unroll=False)

    def step(g, b, tb):
        wait_gather(b)

        @pl.when(g + NBUF < N_GROUPS)
        def _():
            start_idx(g + NBUF, b)

        b2 = (b + 2) % NBUF

        @pl.when(g + 2 < N_GROUPS)
        def _():
            wait_idx(g + 2, b2)
            start_gather(b2)

        @pl.when(g >= 2)
        def _():
            wait_store(g - 2, tb)

        transpose(b, tb)
        start_store(g, tb)

    for g in range(NBUF):
        start_idx(g, g)
    for g in range(2):
        wait_idx(g, g)
        start_gather(g)

    def main_body(i, carry):
        g0 = i * NBUF
        for db in range(NBUF):
            step(g0 + db, db, db % 2)
        return carry

    lax.fori_loop(0, _MAIN // NBUF, main_body, 0, unroll=False)

    for g in range(_MAIN, N_GROUPS):
        step(g, g % NBUF, g % 2)

    for g in range(N_GROUPS - 2, N_GROUPS):
        wait_store(g, g % 2)


def kernel(input_ids, embedding_table):
    flat_t = _detile_idx(input_ids.T)
    o5 = _gather_kernel(embedding_table, flat_t)
    return (o5.transpose(2, 4, 0, 1, 3)
            .reshape(BATCH, MAX_LEN, EMBED_DIM))
